# Initial kernel scaffold; baseline (speedup 1.0000x reference)
import jax
import jax.numpy as jnp
from jax.experimental import pallas as pl


def kernel(nfeat, efeat, edge_index, timestamps, W_self1, W_neigh1, wt1, bt1, W_self2, W_neigh2, wt2, bt2):
    raise NotImplementedError("write your pallas kernel here")


# speedup vs baseline: 2.1844x; 2.1844x over previous
import functools
import jax
import jax.numpy as jnp
from jax import lax
from jax.experimental import pallas as pl
from jax.experimental.pallas import tpu as pltpu
from jax.experimental.pallas import tpu_sc as plsc

N = 10000
NP = 10240
E = 320000
D = 128
DE = 16
T = 16
H = 128
NC, NS = 2, 16
C = 80
ROWS = E // C
RPW1 = ROWS // (NC * NS)
RPW = ROWS // NS
IB = 50
NB = RPW // IB
STRIPE = NP // NS
F32 = jnp.float32


def _mesh():
    return plsc.VectorSubcoreMesh(core_axis_name="c", subcore_axis_name="s")


def _sc_small_segsums(te1, te2, efeat, idxd2, idxs2, z16, ones16):
    out_t = jax.ShapeDtypeStruct((2, NP, NC * 4 * DE), F32)
    scratch = [pltpu.VMEM_SHARED((NP, DE), F32) for _ in range(8)]
    scratch += [
        pltpu.VMEM((RPW1, C), jnp.int32),
        pltpu.VMEM((RPW1, C), jnp.int32),
        pltpu.VMEM((C, DE), F32),
        pltpu.VMEM((C, DE), F32),
        pltpu.VMEM((C, DE), F32),
        pltpu.VMEM((C, DE), F32),
    ]

    @functools.partial(pl.kernel, out_type=out_t, mesh=_mesh(),
                       scratch_types=scratch,
                       compiler_params=pltpu.CompilerParams(
                           use_tc_tiling_on_sc=False))
    def k(te1_h, te2_h, ef_h, idxd_h, idxs_h, z_h, ones_h, out_h,
          a_efd, a_t1d, a_t2d, a_1d, a_efs, a_t1s, a_t2s, a_1s,
          idxd_v, idxs_v, ef_v, t1_v, t2_v, one_v):
        c = lax.axis_index("c")
        s = lax.axis_index("s")
        accs = ((a_efd, a_t1d, a_t2d, a_1d), (a_efs, a_t1s, a_t2s, a_1s))
        for g in range(2):
            for a in range(4):
                pltpu.sync_copy(z_h, accs[g][a].at[pl.ds(s * STRIPE, STRIPE)])
        pltpu.sync_copy(ones_h, one_v)
        w = c * NS + s
        pltpu.sync_copy(idxd_h.at[w], idxd_v)
        pltpu.sync_copy(idxs_h.at[w], idxs_v)
        plsc.subcore_barrier()

        @pl.loop(0, RPW1)
        def _(j):
            erow = (w * RPW1 + j) * C
            pltpu.sync_copy(ef_h.at[pl.ds(erow, C)], ef_v)
            pltpu.sync_copy(te1_h.at[pl.ds(erow, C)], t1_v)
            pltpu.sync_copy(te2_h.at[pl.ds(erow, C)], t2_v)
            pltpu.sync_copy(ef_v, a_efd.at[idxd_v.at[j]], add=True)
            pltpu.sync_copy(ef_v, a_efs.at[idxs_v.at[j]], add=True)
            pltpu.sync_copy(t1_v, a_t1d.at[idxd_v.at[j]], add=True)
            pltpu.sync_copy(t1_v, a_t1s.at[idxs_v.at[j]], add=True)
            pltpu.sync_copy(t2_v, a_t2d.at[idxd_v.at[j]], add=True)
            pltpu.sync_copy(t2_v, a_t2s.at[idxs_v.at[j]], add=True)
            pltpu.sync_copy(one_v, a_1d.at[idxd_v.at[j]], add=True)
            pltpu.sync_copy(one_v, a_1s.at[idxs_v.at[j]], add=True)

        plsc.subcore_barrier()
        for g in range(2):
            for a in range(4):
                pltpu.sync_copy(
                    accs[g][a].at[pl.ds(s * STRIPE, STRIPE)],
                    out_h.at[g, pl.ds(s * STRIPE, STRIPE),
                             pl.ds((c * 4 + a) * DE, DE)])

    return k(te1, te2, efeat, idxd2, idxs2, z16, ones16)


def _sc_spmm(nfeat, gidx, sidx, z128):
    out_t = jax.ShapeDtypeStruct((NC, NP, D), F32)
    scratch = [
        pltpu.VMEM_SHARED((NP, D), F32),
        pltpu.VMEM((IB, C), jnp.int32),
        pltpu.VMEM((IB, C), jnp.int32),
        pltpu.VMEM((C, D), F32),
    ]

    @functools.partial(pl.kernel, out_type=out_t, mesh=_mesh(),
                       scratch_types=scratch,
                       compiler_params=pltpu.CompilerParams(
                           use_tc_tiling_on_sc=False))
    def k(nf_h, gi_h, si_h, z_h, out_h, acc, gi_v, si_v, rows_v):
        c = lax.axis_index("c")
        s = lax.axis_index("s")
        pltpu.sync_copy(z_h, acc.at[pl.ds(s * STRIPE, STRIPE)])
        plsc.subcore_barrier()

        @pl.loop(0, NB)
        def _(b):
            pltpu.sync_copy(gi_h.at[c, s, b], gi_v)
            pltpu.sync_copy(si_h.at[c, s, b], si_v)

            @pl.loop(0, IB)
            def _(j):
                pltpu.sync_copy(nf_h.at[gi_v.at[j]], rows_v)
                pltpu.sync_copy(rows_v, acc.at[si_v.at[j]], add=True)

        plsc.subcore_barrier()
        pltpu.sync_copy(acc.at[pl.ds(s * STRIPE, STRIPE)],
                        out_h.at[c, pl.ds(s * STRIPE, STRIPE)])

    return k(nfeat, gidx, sidx, z128)


def _sc_gather(tab2, gidx):
    out_t = jax.ShapeDtypeStruct((NC, E, D), F32)
    scratch = [
        pltpu.VMEM((RPW, C), jnp.int32),
        pltpu.VMEM((C, D), F32),
    ]

    @functools.partial(pl.kernel, out_type=out_t, mesh=_mesh(),
                       scratch_types=scratch,
                       compiler_params=pltpu.CompilerParams(
                           use_tc_tiling_on_sc=False))
    def k(tab_h, gi_h, out_h, gi_v, rows_v):
        c = lax.axis_index("c")
        s = lax.axis_index("s")
        pltpu.sync_copy(gi_h.at[c, s], gi_v)

        @pl.loop(0, RPW)
        def _(j):
            pltpu.sync_copy(tab_h.at[gi_v.at[j]], rows_v)
            pltpu.sync_copy(rows_v, out_h.at[c, pl.ds((s * RPW + j) * C, C)])

    return k(tab2, gidx)


def _sc_scatter_edges(vals, sidx, z128):
    out_t = jax.ShapeDtypeStruct((NC, NP, D), F32)
    scratch = [
        pltpu.VMEM_SHARED((NP, D), F32),
        pltpu.VMEM((RPW, C), jnp.int32),
        pltpu.VMEM((C, D), F32),
    ]

    @functools.partial(pl.kernel, out_type=out_t, mesh=_mesh(),
                       scratch_types=scratch,
                       compiler_params=pltpu.CompilerParams(
                           use_tc_tiling_on_sc=False))
    def k(v_h, si_h, z_h, out_h, acc, si_v, rows_v):
        c = lax.axis_index("c")
        s = lax.axis_index("s")
        pltpu.sync_copy(z_h, acc.at[pl.ds(s * STRIPE, STRIPE)])
        pltpu.sync_copy(si_h.at[c, s], si_v)
        plsc.subcore_barrier()

        @pl.loop(0, RPW)
        def _(j):
            pltpu.sync_copy(v_h.at[c, pl.ds((s * RPW + j) * C, C)], rows_v)
            pltpu.sync_copy(rows_v, acc.at[si_v.at[j]], add=True)

        plsc.subcore_barrier()
        pltpu.sync_copy(acc.at[pl.ds(s * STRIPE, STRIPE)],
                        out_h.at[c, pl.ds(s * STRIPE, STRIPE)])

    return k(vals, sidx, z128)


def _dot(a, b):
    return jnp.dot(a, b, precision=jax.lax.Precision.HIGHEST,
                   preferred_element_type=F32)


def _tc_te(ts2, w1, b1, w2, b2, interpret=False):
    B = 4000

    def body(ts_ref, w1_ref, b1_ref, w2_ref, b2_ref, o1, o2):
        t = ts_ref[...]
        o1[...] = jnp.cos(t * w1_ref[...] + b1_ref[...])
        o2[...] = jnp.cos(t * w2_ref[...] + b2_ref[...])

    wspec = pl.BlockSpec((1, T), lambda i: (0, 0))
    return pl.pallas_call(
        body,
        grid=(E // B,),
        in_specs=[pl.BlockSpec((B, 1), lambda i: (i, 0)),
                  wspec, wspec, wspec, wspec],
        out_specs=[pl.BlockSpec((B, T), lambda i: (i, 0))] * 2,
        out_shape=[jax.ShapeDtypeStruct((E, T), F32)] * 2,
        interpret=interpret,
    )(ts2, w1, b1, w2, b2)


def _tc_node1(nfeat, G, S, Ws1, Wn1, Wn2, interpret=False):
    def body(nf, g_ref, s_ref, ws1, wn1, wn2, p_out, t2n_out):
        sv = s_ref[0]
        ones_col = sv[:, 3 * DE:4 * DE] + sv[:, 7 * DE:8 * DE]
        inv = 1.0 / jnp.clip(ones_col[:, 0:1], 1.0, None)
        ef = (sv[:, 0:DE] + sv[:, 4 * DE:5 * DE]) * inv
        t1 = (sv[:, DE:2 * DE] + sv[:, 5 * DE:6 * DE]) * inv
        t2 = (sv[:, 2 * DE:3 * DE] + sv[:, 6 * DE:7 * DE]) * inv
        g = g_ref[0] * inv
        ws1v = ws1[...]
        wn1v = wn1[...]
        u = _dot(nf[...], ws1v[:D])
        p = (u + _dot(g, wn1v[:D]) + _dot(ef, wn1v[D:D + DE])
             + _dot(t1, wn1v[D + DE:]))
        p_out[...] = p[None]
        t2n_out[...] = _dot(t2, wn2[...][H:])[None]

    BN = 2048
    return pl.pallas_call(
        body,
        grid=(NC, NP // BN),
        in_specs=[
            pl.BlockSpec((BN, D), lambda c, i: (i, 0)),
            pl.BlockSpec((1, BN, D), lambda c, i: (1 - c, i, 0)),
            pl.BlockSpec((1, BN, NC * 4 * DE), lambda c, i: (1 - c, i, 0)),
            pl.BlockSpec((D + DE, H), lambda c, i: (0, 0)),
            pl.BlockSpec((D + DE + T, H), lambda c, i: (0, 0)),
            pl.BlockSpec((H + T, H), lambda c, i: (0, 0)),
        ],
        out_specs=[pl.BlockSpec((1, BN, D), lambda c, i: (c, i, 0))] * 2,
        out_shape=[jax.ShapeDtypeStruct((NC, NP, D), F32)] * 2,
        interpret=interpret,
    )(nfeat, G, S, Ws1, Wn1, Wn2)


def _tc_combine(gp, efeat, Ws1, interpret=False):
    B = 2000

    def body(gp_ref, ef_ref, ws1, o_ref):
        efp = _dot(ef_ref[...], ws1[...][D:])
        o_ref[...] = jnp.maximum(gp_ref[...] + efp[None], 0.0)

    return pl.pallas_call(
        body,
        grid=(NC, E // B),
        in_specs=[
            pl.BlockSpec((1, B, D), lambda c, i: (c, i, 0)),
            pl.BlockSpec((B, DE), lambda c, i: (i, 0)),
            pl.BlockSpec((D + DE, H), lambda c, i: (0, 0)),
        ],
        out_specs=pl.BlockSpec((1, B, D), lambda c, i: (c, i, 0)),
        out_shape=jax.ShapeDtypeStruct((NC, E, D), F32),
        interpret=interpret,
    )(gp, efeat, Ws1)


def _tc_node2(A, S, T2n, Wn2, interpret=False):
    def body(a_ref, s_ref, t2n_ref, wn2, q_out):
        sv = s_ref[0]
        ones_col = sv[:, 3 * DE:4 * DE] + sv[:, 7 * DE:8 * DE]
        inv = 1.0 / jnp.clip(ones_col[:, 0:1], 1.0, None)
        q = _dot(a_ref[0] * inv, wn2[...][:H]) + t2n_ref[0]
        q_out[...] = q[None]

    BN = 2048
    return pl.pallas_call(
        body,
        grid=(NC, NP // BN),
        in_specs=[
            pl.BlockSpec((1, BN, D), lambda c, i: (1 - c, i, 0)),
            pl.BlockSpec((1, BN, NC * 4 * DE), lambda c, i: (1 - c, i, 0)),
            pl.BlockSpec((1, BN, D), lambda c, i: (c, i, 0)),
            pl.BlockSpec((H + T, H), lambda c, i: (0, 0)),
        ],
        out_specs=pl.BlockSpec((1, BN, D), lambda c, i: (c, i, 0)),
        out_shape=jax.ShapeDtypeStruct((NC, NP, D), F32),
        interpret=interpret,
    )(A, S, T2n, Wn2)


def _tc_final(v1, gq, Ws2, interpret=False):
    B = 2000

    def body(v_ref, gq_ref, ws2, o_ref):
        m = _dot(v_ref[0], ws2[...])
        o_ref[...] = jnp.maximum(m + gq_ref[0], 0.0)[None]

    return pl.pallas_call(
        body,
        grid=(NC, E // B),
        in_specs=[
            pl.BlockSpec((1, B, D), lambda c, i: (c, i, 0)),
            pl.BlockSpec((1, B, D), lambda c, i: (c, i, 0)),
            pl.BlockSpec((H, H), lambda c, i: (0, 0)),
        ],
        out_specs=pl.BlockSpec((1, B, D), lambda c, i: (c, i, 0)),
        out_shape=jax.ShapeDtypeStruct((NC, E, D), F32),
        interpret=interpret,
    )(v1, gq, Ws2)


@jax.jit
def kernel(nfeat, efeat, edge_index, timestamps,
           W_self1, W_neigh1, wt1, bt1,
           W_self2, W_neigh2, wt2, bt2):
    src = edge_index[0]
    dst = edge_index[1]
    src1 = src.reshape(NC * NS, RPW1, C)
    dst1 = dst.reshape(NC * NS, RPW1, C)
    src2 = src.reshape(NS, RPW, C)
    dst2 = dst.reshape(NS, RPW, C)
    ts2 = timestamps.reshape(E, 1)

    te1, te2 = _tc_te(ts2, wt1.reshape(1, T), bt1.reshape(1, T),
                      wt2.reshape(1, T), bt2.reshape(1, T))

    z16 = jnp.zeros((STRIPE, DE), F32)
    z128 = jnp.zeros((STRIPE, D), F32)
    ones16 = jnp.ones((C, DE), F32)

    S = _sc_small_segsums(te1, te2, efeat, dst1, src1, z16, ones16)

    src2b = src.reshape(NS, NB, IB, C)
    dst2b = dst.reshape(NS, NB, IB, C)
    G = _sc_spmm(nfeat, jnp.stack([src2b, dst2b]), jnp.stack([dst2b, src2b]),
                 z128)

    nfeat_p = jnp.pad(nfeat, ((0, NP - N), (0, 0)))
    P, T2n = _tc_node1(nfeat_p, G, S, W_self1, W_neigh1, W_neigh2)

    gidx_off = jnp.stack([src2, dst2 + NP])
    gp = _sc_gather(P.reshape(NC * NP, D), gidx_off)
    v1 = _tc_combine(gp, efeat, W_self1)

    A = _sc_scatter_edges(v1, jnp.stack([dst2, src2]), z128)
    Q = _tc_node2(A, S, T2n, W_neigh2)
    gq = _sc_gather(Q.reshape(NC * NP, D), gidx_off)

    out2 = _tc_final(v1, gq, W_self2)
    return (out2[0], out2[1])


# speedup vs baseline: 2.9588x; 1.3545x over previous
import functools
import jax
import jax.numpy as jnp
from jax import lax
from jax.experimental import pallas as pl
from jax.experimental.pallas import tpu as pltpu
from jax.experimental.pallas import tpu_sc as plsc

N = 10000
NP = 10240
E = 320000
D = 128
DE = 16
T = 16
H = 128
NC, NS = 2, 16
C = 80
ROWS = E // C
RPW1 = ROWS // (NC * NS)
RPW = ROWS // NS
IB = 50
NB = RPW // IB
STRIPE = NP // NS
F32 = jnp.float32

_SC_PARAMS = pltpu.CompilerParams(use_tc_tiling_on_sc=False)


def _mesh():
    return plsc.VectorSubcoreMesh(core_axis_name="c", subcore_axis_name="s")


def _sc_small_segsums(te1, te2, efeat, idx1, z16, ones16):
    out_t = jax.ShapeDtypeStruct((2, NP, NC * 4 * DE), F32)
    scratch = [pltpu.VMEM_SHARED((NP, DE), F32) for _ in range(8)]
    scratch += [
        pltpu.VMEM((RPW1, C), jnp.int32),
        pltpu.VMEM((RPW1, C), jnp.int32),
        pltpu.VMEM((C, DE), F32),
        pltpu.VMEM((C, DE), F32),
        pltpu.VMEM((C, DE), F32),
        pltpu.VMEM((C, DE), F32),
    ]

    @functools.partial(pl.kernel, out_type=out_t, mesh=_mesh(),
                       scratch_types=scratch, compiler_params=_SC_PARAMS)
    def k(te1_h, te2_h, ef_h, idx_h, z_h, ones_h, out_h,
          a_efd, a_t1d, a_t2d, a_1d, a_efs, a_t1s, a_t2s, a_1s,
          idxd_v, idxs_v, ef_v, t1_v, t2_v, one_v):
        c = lax.axis_index("c")
        s = lax.axis_index("s")
        accs = ((a_efd, a_t1d, a_t2d, a_1d), (a_efs, a_t1s, a_t2s, a_1s))
        for g in range(2):
            for a in range(4):
                pltpu.sync_copy(z_h, accs[g][a].at[pl.ds(s * STRIPE, STRIPE)])
        pltpu.sync_copy(ones_h, one_v)
        w = c * NS + s
        pltpu.sync_copy(idx_h.at[1, w], idxd_v)
        pltpu.sync_copy(idx_h.at[0, w], idxs_v)
        plsc.subcore_barrier()

        @pl.loop(0, RPW1)
        def _(j):
            erow = (w * RPW1 + j) * C
            pltpu.sync_copy(ef_h.at[pl.ds(erow, C)], ef_v)
            pltpu.sync_copy(te1_h.at[pl.ds(erow, C)], t1_v)
            pltpu.sync_copy(te2_h.at[pl.ds(erow, C)], t2_v)
            pltpu.sync_copy(ef_v, a_efd.at[idxd_v.at[j]], add=True)
            pltpu.sync_copy(ef_v, a_efs.at[idxs_v.at[j]], add=True)
            pltpu.sync_copy(t1_v, a_t1d.at[idxd_v.at[j]], add=True)
            pltpu.sync_copy(t1_v, a_t1s.at[idxs_v.at[j]], add=True)
            pltpu.sync_copy(t2_v, a_t2d.at[idxd_v.at[j]], add=True)
            pltpu.sync_copy(t2_v, a_t2s.at[idxs_v.at[j]], add=True)
            pltpu.sync_copy(one_v, a_1d.at[idxd_v.at[j]], add=True)
            pltpu.sync_copy(one_v, a_1s.at[idxs_v.at[j]], add=True)

        plsc.subcore_barrier()
        for g in range(2):
            for a in range(4):
                pltpu.sync_copy(
                    accs[g][a].at[pl.ds(s * STRIPE, STRIPE)],
                    out_h.at[g, pl.ds(s * STRIPE, STRIPE),
                             pl.ds((c * 4 + a) * DE, DE)])

    return k(te1, te2, efeat, idx1, z16, ones16)


def _sc_spmm(nfeat, idx2, z128):
    out_t = jax.ShapeDtypeStruct((NC, NP, D), F32)
    scratch = [
        pltpu.VMEM_SHARED((NP, D), F32),
        pltpu.VMEM((IB, C), jnp.int32),
        pltpu.VMEM((IB, C), jnp.int32),
        pltpu.VMEM((C, D), F32),
    ]

    @functools.partial(pl.kernel, out_type=out_t, mesh=_mesh(),
                       scratch_types=scratch, compiler_params=_SC_PARAMS)
    def k(nf_h, idx_h, z_h, out_h, acc, gi_v, si_v, rows_v):
        c = lax.axis_index("c")
        s = lax.axis_index("s")
        pltpu.sync_copy(z_h, acc.at[pl.ds(s * STRIPE, STRIPE)])
        plsc.subcore_barrier()

        @pl.loop(0, NB)
        def _(b):
            pltpu.sync_copy(idx_h.at[c, s, b], gi_v)
            pltpu.sync_copy(idx_h.at[1 - c, s, b], si_v)

            @pl.loop(0, IB)
            def _(j):
                pltpu.sync_copy(nf_h.at[gi_v.at[j]], rows_v)
                pltpu.sync_copy(rows_v, acc.at[si_v.at[j]], add=True)

        plsc.subcore_barrier()
        pltpu.sync_copy(acc.at[pl.ds(s * STRIPE, STRIPE)],
                        out_h.at[c, pl.ds(s * STRIPE, STRIPE)])

    return k(nfeat, idx2, z128)


def _sc_gather(tab, idx4):
    out_t = jax.ShapeDtypeStruct((NC, E, D), F32)
    scratch = [
        pltpu.VMEM((RPW, C), jnp.int32),
        pltpu.VMEM((C, D), F32),
    ]

    @functools.partial(pl.kernel, out_type=out_t, mesh=_mesh(),
                       scratch_types=scratch, compiler_params=_SC_PARAMS)
    def k(tab_h, idx_h, out_h, gi_v, rows_v):
        c = lax.axis_index("c")
        s = lax.axis_index("s")
        pltpu.sync_copy(idx_h.at[c, s], gi_v)

        @pl.loop(0, RPW)
        def _(j):
            pltpu.sync_copy(tab_h.at[c].at[gi_v.at[j]], rows_v)
            pltpu.sync_copy(rows_v, out_h.at[c, pl.ds((s * RPW + j) * C, C)])

    return k(tab, idx4)


def _sc_scatter_edges(vals, idx4, z128):
    out_t = jax.ShapeDtypeStruct((NC, NP, D), F32)
    scratch = [
        pltpu.VMEM_SHARED((NP, D), F32),
        pltpu.VMEM((RPW, C), jnp.int32),
        pltpu.VMEM((C, D), F32),
    ]

    @functools.partial(pl.kernel, out_type=out_t, mesh=_mesh(),
                       scratch_types=scratch, compiler_params=_SC_PARAMS)
    def k(v_h, idx_h, z_h, out_h, acc, si_v, rows_v):
        c = lax.axis_index("c")
        s = lax.axis_index("s")
        pltpu.sync_copy(z_h, acc.at[pl.ds(s * STRIPE, STRIPE)])
        pltpu.sync_copy(idx_h.at[1 - c, s], si_v)
        plsc.subcore_barrier()

        @pl.loop(0, RPW)
        def _(j):
            pltpu.sync_copy(v_h.at[c, pl.ds((s * RPW + j) * C, C)], rows_v)
            pltpu.sync_copy(rows_v, acc.at[si_v.at[j]], add=True)

        plsc.subcore_barrier()
        pltpu.sync_copy(acc.at[pl.ds(s * STRIPE, STRIPE)],
                        out_h.at[c, pl.ds(s * STRIPE, STRIPE)])

    return k(vals, idx4, z128)


def _dot(a, b):
    return jnp.dot(a, b, precision=jax.lax.Precision.HIGHEST,
                   preferred_element_type=F32)


def _tc_te(tsx, w1t, b1t, w2t, b2t, interpret=False):
    ET8 = E * T // 128
    BR = 4000

    def body(ts_ref, w1_ref, b1_ref, w2_ref, b2_ref, o1, o2):
        t = ts_ref[...]
        o1[...] = jnp.cos(t * w1_ref[...] + b1_ref[...])
        o2[...] = jnp.cos(t * w2_ref[...] + b2_ref[...])

    wspec = pl.BlockSpec((1, 128), lambda i: (0, 0))
    return pl.pallas_call(
        body,
        grid=(ET8 // BR,),
        in_specs=[pl.BlockSpec((BR, 128), lambda i: (i, 0)),
                  wspec, wspec, wspec, wspec],
        out_specs=[pl.BlockSpec((BR, 128), lambda i: (i, 0))] * 2,
        out_shape=[jax.ShapeDtypeStruct((ET8, 128), F32)] * 2,
        compiler_params=pltpu.CompilerParams(
            dimension_semantics=("parallel",)),
        interpret=interpret,
    )(tsx, w1t, b1t, w2t, b2t)


def _tc_node1(nfeat, G, S, Ws1, Wn1, Wn2, interpret=False):
    def body(nf, g_ref, s_ref, ws1, wn1, wn2, p_out, t2n_out):
        sv = s_ref[0]
        ones_col = sv[:, 3 * DE:4 * DE] + sv[:, 7 * DE:8 * DE]
        inv = 1.0 / jnp.clip(ones_col[:, 0:1], 1.0, None)
        ef = (sv[:, 0:DE] + sv[:, 4 * DE:5 * DE]) * inv
        t1 = (sv[:, DE:2 * DE] + sv[:, 5 * DE:6 * DE]) * inv
        t2 = (sv[:, 2 * DE:3 * DE] + sv[:, 6 * DE:7 * DE]) * inv
        g = g_ref[0] * inv
        ws1v = ws1[...]
        wn1v = wn1[...]
        u = _dot(nf[...], ws1v[:D])
        p = (u + _dot(g, wn1v[:D]) + _dot(ef, wn1v[D:D + DE])
             + _dot(t1, wn1v[D + DE:]))
        p_out[...] = p[None]
        t2n_out[...] = _dot(t2, wn2[...][H:])[None]

    BN = 2048
    return pl.pallas_call(
        body,
        grid=(NC, NP // BN),
        in_specs=[
            pl.BlockSpec((BN, D), lambda c, i: (i, 0)),
            pl.BlockSpec((1, BN, D), lambda c, i: (1 - c, i, 0)),
            pl.BlockSpec((1, BN, NC * 4 * DE), lambda c, i: (1 - c, i, 0)),
            pl.BlockSpec((D + DE, H), lambda c, i: (0, 0)),
            pl.BlockSpec((D + DE + T, H), lambda c, i: (0, 0)),
            pl.BlockSpec((H + T, H), lambda c, i: (0, 0)),
        ],
        out_specs=[pl.BlockSpec((1, BN, D), lambda c, i: (c, i, 0))] * 2,
        out_shape=[jax.ShapeDtypeStruct((NC, NP, D), F32)] * 2,
        compiler_params=pltpu.CompilerParams(
            dimension_semantics=("parallel", "parallel")),
        interpret=interpret,
    )(nfeat, G, S, Ws1, Wn1, Wn2)


def _tc_combine(gp, efeat, Ws1, interpret=False):
    B = 2000

    def body(gp_ref, ef_ref, ws1, o_ref):
        efp = _dot(ef_ref[...], ws1[...][D:])
        o_ref[...] = jnp.maximum(gp_ref[...] + efp[None], 0.0)

    return pl.pallas_call(
        body,
        grid=(NC, E // B),
        in_specs=[
            pl.BlockSpec((1, B, D), lambda c, i: (c, i, 0)),
            pl.BlockSpec((B, DE), lambda c, i: (i, 0)),
            pl.BlockSpec((D + DE, H), lambda c, i: (0, 0)),
        ],
        out_specs=pl.BlockSpec((1, B, D), lambda c, i: (c, i, 0)),
        out_shape=jax.ShapeDtypeStruct((NC, E, D), F32),
        compiler_params=pltpu.CompilerParams(
            dimension_semantics=("parallel", "parallel")),
        interpret=interpret,
    )(gp, efeat, Ws1)


def _tc_node2(A, S, T2n, Wn2, interpret=False):
    def body(a_ref, s_ref, t2n_ref, wn2, q_out):
        sv = s_ref[0]
        ones_col = sv[:, 3 * DE:4 * DE] + sv[:, 7 * DE:8 * DE]
        inv = 1.0 / jnp.clip(ones_col[:, 0:1], 1.0, None)
        q = _dot(a_ref[0] * inv, wn2[...][:H]) + t2n_ref[0]
        q_out[...] = q[None]

    BN = 2048
    return pl.pallas_call(
        body,
        grid=(NC, NP // BN),
        in_specs=[
            pl.BlockSpec((1, BN, D), lambda c, i: (1 - c, i, 0)),
            pl.BlockSpec((1, BN, NC * 4 * DE), lambda c, i: (1 - c, i, 0)),
            pl.BlockSpec((1, BN, D), lambda c, i: (c, i, 0)),
            pl.BlockSpec((H + T, H), lambda c, i: (0, 0)),
        ],
        out_specs=pl.BlockSpec((1, BN, D), lambda c, i: (c, i, 0)),
        out_shape=jax.ShapeDtypeStruct((NC, NP, D), F32),
        compiler_params=pltpu.CompilerParams(
            dimension_semantics=("parallel", "parallel")),
        interpret=interpret,
    )(A, S, T2n, Wn2)


def _tc_final(v1, gq, Ws2, interpret=False):
    B = 2000

    def body(v_ref, gq_ref, ws2, o_ref):
        m = _dot(v_ref[0], ws2[...])
        o_ref[...] = jnp.maximum(m + gq_ref[0], 0.0)[None]

    return pl.pallas_call(
        body,
        grid=(NC, E // B),
        in_specs=[
            pl.BlockSpec((1, B, D), lambda c, i: (c, i, 0)),
            pl.BlockSpec((1, B, D), lambda c, i: (c, i, 0)),
            pl.BlockSpec((H, H), lambda c, i: (0, 0)),
        ],
        out_specs=pl.BlockSpec((1, B, D), lambda c, i: (c, i, 0)),
        out_shape=jax.ShapeDtypeStruct((NC, E, D), F32),
        compiler_params=pltpu.CompilerParams(
            dimension_semantics=("parallel", "parallel")),
        interpret=interpret,
    )(v1, gq, Ws2)


@jax.jit
def kernel(nfeat, efeat, edge_index, timestamps,
           W_self1, W_neigh1, wt1, bt1,
           W_self2, W_neigh2, wt2, bt2):
    idx = jnp.stack([edge_index[0], edge_index[1]])
    idx1 = idx.reshape(2, NC * NS, RPW1, C)
    idx2 = idx.reshape(2, NS, NB, IB, C)
    idx4 = idx.reshape(2, NS, RPW, C)

    tsx = jnp.broadcast_to(timestamps[:, None], (E, T)).reshape(E * T // 128,
                                                                128)
    tile = lambda v: jnp.tile(v, 128 // T).reshape(1, 128)
    te1f, te2f = _tc_te(tsx, tile(wt1), tile(bt1), tile(wt2), tile(bt2))
    te1 = te1f.reshape(E, T)
    te2 = te2f.reshape(E, T)

    z16 = jnp.zeros((STRIPE, DE), F32)
    z128 = jnp.zeros((STRIPE, D), F32)
    ones16 = jnp.ones((C, DE), F32)

    S = _sc_small_segsums(te1, te2, efeat, idx1, z16, ones16)
    G = _sc_spmm(nfeat, idx2, z128)

    nfeat_p = jnp.pad(nfeat, ((0, NP - N), (0, 0)))
    P, T2n = _tc_node1(nfeat_p, G, S, W_self1, W_neigh1, W_neigh2)

    gp = _sc_gather(P, idx4)
    v1 = _tc_combine(gp, efeat, W_self1)

    A = _sc_scatter_edges(v1, idx4, z128)
    Q = _tc_node2(A, S, T2n, W_neigh2)
    gq = _sc_gather(Q, idx4)

    out2 = _tc_final(v1, gq, W_self2)
    return (out2[0], out2[1])


# speedup vs baseline: 3.6154x; 1.2219x over previous
import functools
import jax
import jax.numpy as jnp
from jax import lax
from jax.experimental import pallas as pl
from jax.experimental.pallas import tpu as pltpu
from jax.experimental.pallas import tpu_sc as plsc

N = 10000
NP = 10240
E = 320000
D = 128
DE = 16
T = 16
H = 128
NC, NS = 2, 16
C = 80
ROWS = E // C
RPW1 = ROWS // (NC * NS)
RPW = ROWS // NS
IB = 50
NB = RPW // IB
STRIPE = NP // NS
F32 = jnp.float32

_SC_PARAMS = pltpu.CompilerParams(use_tc_tiling_on_sc=False)


def _mesh():
    return plsc.VectorSubcoreMesh(core_axis_name="c", subcore_axis_name="s")


def _sc_small_segsums(te1, te2, efeat, idx1, z16, ones16):
    out_t = jax.ShapeDtypeStruct((2, NP, NC * 4 * DE), F32)
    scratch = [pltpu.VMEM_SHARED((NP, DE), F32) for _ in range(8)]
    scratch += [
        pltpu.VMEM((RPW1, C), jnp.int32),
        pltpu.VMEM((RPW1, C), jnp.int32),
        pltpu.VMEM((C, DE), F32),
        pltpu.VMEM((C, DE), F32),
        pltpu.VMEM((C, DE), F32),
        pltpu.VMEM((C, DE), F32),
    ]

    @functools.partial(pl.kernel, out_type=out_t, mesh=_mesh(),
                       scratch_types=scratch, compiler_params=_SC_PARAMS)
    def k(te1_h, te2_h, ef_h, idx_h, z_h, ones_h, out_h,
          a_efd, a_t1d, a_t2d, a_1d, a_efs, a_t1s, a_t2s, a_1s,
          idxd_v, idxs_v, ef_v, t1_v, t2_v, one_v):
        c = lax.axis_index("c")
        s = lax.axis_index("s")
        accs = ((a_efd, a_t1d, a_t2d, a_1d), (a_efs, a_t1s, a_t2s, a_1s))
        for g in range(2):
            for a in range(4):
                pltpu.sync_copy(z_h, accs[g][a].at[pl.ds(s * STRIPE, STRIPE)])
        pltpu.sync_copy(ones_h, one_v)
        w = c * NS + s
        pltpu.sync_copy(idx_h.at[1, w], idxd_v)
        pltpu.sync_copy(idx_h.at[0, w], idxs_v)
        plsc.subcore_barrier()

        @pl.loop(0, RPW1)
        def _(j):
            erow = (w * RPW1 + j) * C
            pltpu.sync_copy(ef_h.at[pl.ds(erow, C)], ef_v)
            pltpu.sync_copy(te1_h.at[pl.ds(erow, C)], t1_v)
            pltpu.sync_copy(te2_h.at[pl.ds(erow, C)], t2_v)
            pltpu.sync_copy(ef_v, a_efd.at[idxd_v.at[j]], add=True)
            pltpu.sync_copy(ef_v, a_efs.at[idxs_v.at[j]], add=True)
            pltpu.sync_copy(t1_v, a_t1d.at[idxd_v.at[j]], add=True)
            pltpu.sync_copy(t1_v, a_t1s.at[idxs_v.at[j]], add=True)
            pltpu.sync_copy(t2_v, a_t2d.at[idxd_v.at[j]], add=True)
            pltpu.sync_copy(t2_v, a_t2s.at[idxs_v.at[j]], add=True)
            pltpu.sync_copy(one_v, a_1d.at[idxd_v.at[j]], add=True)
            pltpu.sync_copy(one_v, a_1s.at[idxs_v.at[j]], add=True)

        plsc.subcore_barrier()
        for g in range(2):
            for a in range(4):
                pltpu.sync_copy(
                    accs[g][a].at[pl.ds(s * STRIPE, STRIPE)],
                    out_h.at[g, pl.ds(s * STRIPE, STRIPE),
                             pl.ds((c * 4 + a) * DE, DE)])

    return k(te1, te2, efeat, idx1, z16, ones16)


def _sc_spmm(nfeat, idx2, z128):
    out_t = jax.ShapeDtypeStruct((NC, NP, D), F32)
    scratch = [
        pltpu.VMEM_SHARED((NP, D), F32),
        pltpu.VMEM((IB, C), jnp.int32),
        pltpu.VMEM((IB, C), jnp.int32),
        pltpu.VMEM((C, D), F32),
    ]

    @functools.partial(pl.kernel, out_type=out_t, mesh=_mesh(),
                       scratch_types=scratch, compiler_params=_SC_PARAMS)
    def k(nf_h, idx_h, z_h, out_h, acc, gi_v, si_v, rows_v):
        c = lax.axis_index("c")
        s = lax.axis_index("s")
        pltpu.sync_copy(z_h, acc.at[pl.ds(s * STRIPE, STRIPE)])
        plsc.subcore_barrier()

        @pl.loop(0, NB)
        def _(b):
            pltpu.sync_copy(idx_h.at[c, s, b], gi_v)
            pltpu.sync_copy(idx_h.at[1 - c, s, b], si_v)

            @pl.loop(0, IB)
            def _(j):
                pltpu.sync_copy(nf_h.at[gi_v.at[j]], rows_v)
                pltpu.sync_copy(rows_v, acc.at[si_v.at[j]], add=True)

        plsc.subcore_barrier()
        pltpu.sync_copy(acc.at[pl.ds(s * STRIPE, STRIPE)],
                        out_h.at[c, pl.ds(s * STRIPE, STRIPE)])

    return k(nfeat, idx2, z128)


def _sc_gather(tab, idx4):
    out_t = jax.ShapeDtypeStruct((NC, E, D), F32)
    scratch = [
        pltpu.VMEM((RPW, C), jnp.int32),
        pltpu.VMEM((C, D), F32),
    ]

    @functools.partial(pl.kernel, out_type=out_t, mesh=_mesh(),
                       scratch_types=scratch, compiler_params=_SC_PARAMS)
    def k(tab_h, idx_h, out_h, gi_v, rows_v):
        c = lax.axis_index("c")
        s = lax.axis_index("s")
        pltpu.sync_copy(idx_h.at[c, s], gi_v)

        @pl.loop(0, RPW)
        def _(j):
            pltpu.sync_copy(tab_h.at[c].at[gi_v.at[j]], rows_v)
            pltpu.sync_copy(rows_v, out_h.at[c, pl.ds((s * RPW + j) * C, C)])

    return k(tab, idx4)


def _sc_scatter_edges(vals, idx4, z128):
    out_t = jax.ShapeDtypeStruct((NC, NP, D), F32)
    scratch = [
        pltpu.VMEM_SHARED((NP, D), F32),
        pltpu.VMEM((RPW, C), jnp.int32),
        pltpu.VMEM((C, D), F32),
    ]

    @functools.partial(pl.kernel, out_type=out_t, mesh=_mesh(),
                       scratch_types=scratch, compiler_params=_SC_PARAMS)
    def k(v_h, idx_h, z_h, out_h, acc, si_v, rows_v):
        c = lax.axis_index("c")
        s = lax.axis_index("s")
        pltpu.sync_copy(z_h, acc.at[pl.ds(s * STRIPE, STRIPE)])
        pltpu.sync_copy(idx_h.at[1 - c, s], si_v)
        plsc.subcore_barrier()

        @pl.loop(0, RPW)
        def _(j):
            pltpu.sync_copy(v_h.at[c, pl.ds((s * RPW + j) * C, C)], rows_v)
            pltpu.sync_copy(rows_v, acc.at[si_v.at[j]], add=True)

        plsc.subcore_barrier()
        pltpu.sync_copy(acc.at[pl.ds(s * STRIPE, STRIPE)],
                        out_h.at[c, pl.ds(s * STRIPE, STRIPE)])

    return k(vals, idx4, z128)


def _dot(a, b):
    return jnp.dot(a, b, precision=jax.lax.Precision.HIGHEST,
                   preferred_element_type=F32)


def _dot_e(a, b):
    return jnp.dot(a, b, preferred_element_type=F32)


def _tc_te(tsx, w1t, b1t, w2t, b2t, interpret=False):
    ET8 = E * T // 128
    BR = 4000

    def body(ts_ref, w1_ref, b1_ref, w2_ref, b2_ref, o1, o2):
        t = ts_ref[...]
        o1[...] = jnp.cos(t * w1_ref[...] + b1_ref[...])
        o2[...] = jnp.cos(t * w2_ref[...] + b2_ref[...])

    wspec = pl.BlockSpec((1, 128), lambda i: (0, 0))
    return pl.pallas_call(
        body,
        grid=(ET8 // BR,),
        in_specs=[pl.BlockSpec((BR, 128), lambda i: (i, 0)),
                  wspec, wspec, wspec, wspec],
        out_specs=[pl.BlockSpec((BR, 128), lambda i: (i, 0))] * 2,
        out_shape=[jax.ShapeDtypeStruct((ET8, 128), F32)] * 2,
        compiler_params=pltpu.CompilerParams(
            dimension_semantics=("parallel",)),
        interpret=interpret,
    )(tsx, w1t, b1t, w2t, b2t)


def _tc_node1(nfeat, G, S, Ws1, Wn1, Wn2, interpret=False):
    def body(nf, g_ref, s_ref, ws1, wn1, wn2, p_out, t2n_out):
        sv = s_ref[0]
        ones_col = sv[:, 3 * DE:4 * DE] + sv[:, 7 * DE:8 * DE]
        inv = 1.0 / jnp.clip(ones_col[:, 0:1], 1.0, None)
        ef = (sv[:, 0:DE] + sv[:, 4 * DE:5 * DE]) * inv
        t1 = (sv[:, DE:2 * DE] + sv[:, 5 * DE:6 * DE]) * inv
        t2 = (sv[:, 2 * DE:3 * DE] + sv[:, 6 * DE:7 * DE]) * inv
        g = g_ref[0] * inv
        ws1v = ws1[...]
        wn1v = wn1[...]
        u = _dot(nf[...], ws1v[:D])
        p = (u + _dot(g, wn1v[:D]) + _dot(ef, wn1v[D:D + DE])
             + _dot(t1, wn1v[D + DE:]))
        p_out[...] = p[None]
        t2n_out[...] = _dot(t2, wn2[...][H:])[None]

    BN = 2048
    return pl.pallas_call(
        body,
        grid=(NC, NP // BN),
        in_specs=[
            pl.BlockSpec((BN, D), lambda c, i: (i, 0)),
            pl.BlockSpec((1, BN, D), lambda c, i: (1 - c, i, 0)),
            pl.BlockSpec((1, BN, NC * 4 * DE), lambda c, i: (1 - c, i, 0)),
            pl.BlockSpec((D + DE, H), lambda c, i: (0, 0)),
            pl.BlockSpec((D + DE + T, H), lambda c, i: (0, 0)),
            pl.BlockSpec((H + T, H), lambda c, i: (0, 0)),
        ],
        out_specs=[pl.BlockSpec((1, BN, D), lambda c, i: (c, i, 0))] * 2,
        out_shape=[jax.ShapeDtypeStruct((NC, NP, D), F32)] * 2,
        compiler_params=pltpu.CompilerParams(
            dimension_semantics=("parallel", "parallel")),
        interpret=interpret,
    )(nfeat, G, S, Ws1, Wn1, Wn2)


def _tc_combine(gp, efeat, Ws1, interpret=False):
    B = 2000

    def body(gp_ref, ef_ref, ws1, o_ref):
        efp = _dot_e(ef_ref[...], ws1[...][D:])
        o_ref[...] = jnp.maximum(gp_ref[...] + efp[None], 0.0)

    return pl.pallas_call(
        body,
        grid=(E // B,),
        in_specs=[
            pl.BlockSpec((NC, B, D), lambda i: (0, i, 0)),
            pl.BlockSpec((B, DE), lambda i: (i, 0)),
            pl.BlockSpec((D + DE, H), lambda i: (0, 0)),
        ],
        out_specs=pl.BlockSpec((NC, B, D), lambda i: (0, i, 0)),
        out_shape=jax.ShapeDtypeStruct((NC, E, D), F32),
        compiler_params=pltpu.CompilerParams(
            dimension_semantics=("parallel",)),
        interpret=interpret,
    )(gp, efeat, Ws1)


def _tc_node2(A, S, T2n, Wn2, interpret=False):
    def body(a_ref, s_ref, t2n_ref, wn2, q_out):
        sv = s_ref[0]
        ones_col = sv[:, 3 * DE:4 * DE] + sv[:, 7 * DE:8 * DE]
        inv = 1.0 / jnp.clip(ones_col[:, 0:1], 1.0, None)
        q = _dot(a_ref[0] * inv, wn2[...][:H]) + t2n_ref[0]
        q_out[...] = q[None]

    BN = 2048
    return pl.pallas_call(
        body,
        grid=(NC, NP // BN),
        in_specs=[
            pl.BlockSpec((1, BN, D), lambda c, i: (1 - c, i, 0)),
            pl.BlockSpec((1, BN, NC * 4 * DE), lambda c, i: (1 - c, i, 0)),
            pl.BlockSpec((1, BN, D), lambda c, i: (c, i, 0)),
            pl.BlockSpec((H + T, H), lambda c, i: (0, 0)),
        ],
        out_specs=pl.BlockSpec((1, BN, D), lambda c, i: (c, i, 0)),
        out_shape=jax.ShapeDtypeStruct((NC, NP, D), F32),
        compiler_params=pltpu.CompilerParams(
            dimension_semantics=("parallel", "parallel")),
        interpret=interpret,
    )(A, S, T2n, Wn2)


def _tc_final(v1, gq, Ws2, interpret=False):
    B = 2000

    def body(v_ref, gq_ref, ws2, s_ref, d_ref):
        w = ws2[...]
        s_ref[...] = jnp.maximum(_dot_e(v_ref[0], w) + gq_ref[0], 0.0)
        d_ref[...] = jnp.maximum(_dot_e(v_ref[1], w) + gq_ref[1], 0.0)

    return pl.pallas_call(
        body,
        grid=(E // B,),
        in_specs=[
            pl.BlockSpec((NC, B, D), lambda i: (0, i, 0)),
            pl.BlockSpec((NC, B, D), lambda i: (0, i, 0)),
            pl.BlockSpec((H, H), lambda i: (0, 0)),
        ],
        out_specs=[pl.BlockSpec((B, D), lambda i: (i, 0))] * 2,
        out_shape=[jax.ShapeDtypeStruct((E, D), F32)] * 2,
        compiler_params=pltpu.CompilerParams(
            dimension_semantics=("parallel",)),
        interpret=interpret,
    )(v1, gq, Ws2)


@jax.jit
def kernel(nfeat, efeat, edge_index, timestamps,
           W_self1, W_neigh1, wt1, bt1,
           W_self2, W_neigh2, wt2, bt2):
    idx = jnp.stack([edge_index[0], edge_index[1]])
    idx1 = idx.reshape(2, NC * NS, RPW1, C)
    idx2 = idx.reshape(2, NS, NB, IB, C)
    idx4 = idx.reshape(2, NS, RPW, C)

    tsx = jnp.broadcast_to(timestamps[:, None], (E, T)).reshape(E * T // 128,
                                                                128)
    tile = lambda v: jnp.tile(v, 128 // T).reshape(1, 128)
    te1f, te2f = _tc_te(tsx, tile(wt1), tile(bt1), tile(wt2), tile(bt2))
    te1 = te1f.reshape(E, T)
    te2 = te2f.reshape(E, T)

    z16 = jnp.zeros((STRIPE, DE), F32)
    z128 = jnp.zeros((STRIPE, D), F32)
    ones16 = jnp.ones((C, DE), F32)

    S = _sc_small_segsums(te1, te2, efeat, idx1, z16, ones16)
    G = _sc_spmm(nfeat, idx2, z128)

    nfeat_p = jnp.pad(nfeat, ((0, NP - N), (0, 0)))
    P, T2n = _tc_node1(nfeat_p, G, S, W_self1, W_neigh1, W_neigh2)

    gp = _sc_gather(P, idx4)
    v1 = _tc_combine(gp, efeat, W_self1)

    A = _sc_scatter_edges(v1, idx4, z128)
    Q = _tc_node2(A, S, T2n, W_neigh2)
    gq = _sc_gather(Q, idx4)

    s2, d2 = _tc_final(v1, gq, W_self2)
    return (s2, d2)
